# baseline (speedup 1.0000x reference)
import functools

import jax
import jax.numpy as jnp
from jax import lax
from jax.experimental import pallas as pl
from jax.experimental.pallas import tpu as pltpu
from jax.experimental.pallas import tpu_sc as plsc

N_ROWS = 10000
HIDDEN = 256
NUM_SEGS = 64
NUM_CLASSES = 10

NC = 2
NS = 16
L = 16
NW = NC * NS
CHUNK = (N_ROWS // NW) // 8 * 8
TAIL = N_ROWS - CHUNK * NW
TAIL_PER = 8
NCOL = HIDDEN // L


def _sc_pool_kernel(x_hbm, ids_hbm, sums_hbm, cnts_hbm,
                    x_v, ids_v, x_tail_v, ids_tail_v, acc_v, cnt_v):
    cid = lax.axis_index("c")
    sid = lax.axis_index("s")
    wid = cid * NS + sid
    base = wid * CHUNK

    zeros16 = jnp.zeros((L,), jnp.float32)
    ones16 = jnp.ones((L,), jnp.float32)

    @pl.loop(0, NUM_SEGS)
    def _zero(r):
        for c in range(NCOL):
            acc_v[r, pl.ds(c * L, L)] = zeros16
        cnt_v[r, :] = zeros16

    pltpu.sync_copy(ids_hbm.at[pl.ds(base, CHUNK)], ids_v)
    pltpu.sync_copy(x_hbm.at[pl.ds(base, CHUNK)], x_v)

    @pl.loop(0, CHUNK // L)
    def _accum(g):
        ids16 = ids_v[pl.ds(g * L, L)]
        for j in range(L):
            seg = ids16[j]
            r = g * L + j
            for c in range(NCOL):
                plsc.addupdate(acc_v.at[seg, pl.ds(c * L, L)],
                               x_v[r, pl.ds(c * L, L)])
            plsc.addupdate(cnt_v.at[seg], ones16)

    if CHUNK % L:
        ids16_t = ids_v[pl.ds(CHUNK - L, L)]
        for j in range(L - CHUNK % L, L):
            seg = ids16_t[j]
            r = CHUNK - L + j
            for c in range(NCOL):
                plsc.addupdate(acc_v.at[seg, pl.ds(c * L, L)],
                               x_v[r, pl.ds(c * L, L)])
            plsc.addupdate(cnt_v.at[seg], ones16)

    @pl.when(wid < TAIL // TAIL_PER)
    def _tail():
        tbase = CHUNK * NW + wid * TAIL_PER
        pltpu.sync_copy(ids_hbm.at[pl.ds(tbase, TAIL_PER)],
                        ids_tail_v.at[pl.ds(0, TAIL_PER)])
        pltpu.sync_copy(x_hbm.at[pl.ds(tbase, TAIL_PER)], x_tail_v)

        ids16_x = ids_tail_v[...]
        for j in range(TAIL_PER):
            seg = ids16_x[j]
            for c in range(NCOL):
                plsc.addupdate(acc_v.at[seg, pl.ds(c * L, L)],
                               x_tail_v[j, pl.ds(c * L, L)])
            plsc.addupdate(cnt_v.at[seg], ones16)

    pltpu.sync_copy(acc_v, sums_hbm.at[wid])
    pltpu.sync_copy(cnt_v, cnts_hbm.at[wid])


def _mlp_head_kernel(sums_ref, cnts_ref, w1_ref, b1_ref, w2_ref, b2_ref,
                     out_ref):
    sums = jnp.sum(sums_ref[...], axis=0)
    cnts = jnp.sum(cnts_ref[...], axis=0)[:, :1]
    pool = sums / jnp.maximum(cnts, 1.0)
    h = jax.lax.dot(pool, w1_ref[...],
                    precision=jax.lax.Precision.HIGHEST,
                    preferred_element_type=jnp.float32)
    h = jnp.maximum(h + b1_ref[...], 0.0)
    logits = jax.lax.dot(h, w2_ref[...],
                         precision=jax.lax.Precision.HIGHEST,
                         preferred_element_type=jnp.float32)
    out_ref[...] = logits + b2_ref[...]


@jax.jit
def _run(x_e, batch_node, W1, b1, W2, b2):
    ids32 = batch_node.astype(jnp.int32)

    sc_pool = pl.kernel(
        _sc_pool_kernel,
        out_type=[
            jax.ShapeDtypeStruct((NW, NUM_SEGS, HIDDEN), jnp.float32),
            jax.ShapeDtypeStruct((NW, NUM_SEGS, L), jnp.float32),
        ],
        mesh=plsc.VectorSubcoreMesh(core_axis_name="c", subcore_axis_name="s"),
        scratch_types=[
            pltpu.VMEM((CHUNK, HIDDEN), jnp.float32),
            pltpu.VMEM((CHUNK,), jnp.int32),
            pltpu.VMEM((TAIL_PER, HIDDEN), jnp.float32),
            pltpu.VMEM((L,), jnp.int32),
            pltpu.VMEM((NUM_SEGS, HIDDEN), jnp.float32),
            pltpu.VMEM((NUM_SEGS, L), jnp.float32),
        ],
    )
    sums, cnts = sc_pool(x_e, ids32)

    b1r = b1.reshape(1, HIDDEN // 2)
    b2r = b2.reshape(1, NUM_CLASSES)
    logits = pl.pallas_call(
        _mlp_head_kernel,
        out_shape=jax.ShapeDtypeStruct((NUM_SEGS, NUM_CLASSES), jnp.float32),
    )(sums, cnts, W1, b1r, W2, b2r)
    return logits


def kernel(x_e, pos_e, edge_index_e, edge_attr_e, batch_node, batch_edge,
           W1, b1, W2, b2):
    return _run(x_e, batch_node, W1, b1, W2, b2)
